# Initial kernel scaffold; baseline (speedup 1.0000x reference)
import jax
import jax.numpy as jnp
from jax.experimental import pallas as pl


def kernel(x, noise):
    raise NotImplementedError("write your pallas kernel here")


# speedup vs baseline: 12.5428x; 12.5428x over previous
import jax
import jax.numpy as jnp
from jax import lax
from jax.experimental import pallas as pl

C = 32
D = 512
N = 250
K = 4
SIGMA = 0.05


def _tc_body(xr_ref, xc_ref, noise_ref, y_ref):
    x_row = xr_ref[0]
    x_col = xc_ref[0]
    ii = lax.broadcasted_iota(jnp.int32, (D, D), 0)
    jj = lax.broadcasted_iota(jnp.int32, (D, D), 1)

    m = (x_row > x_col) | ((x_row == x_col) & (jj < ii))
    rank_col = jnp.sum(m.astype(jnp.int32), axis=1, keepdims=True)

    a2 = (rank_col == jj).astype(jnp.float32)
    x_sorted_row = jnp.sum(a2 * x_col, axis=0, keepdims=True)

    noisy = noise_ref[0] * SIGMA + x_sorted_row
    iota_d = lax.broadcasted_iota(jnp.int32, (N, D), 1)
    neg = jnp.float32(-jnp.inf)
    idxs = []
    for _ in range(K):
        mx = jnp.max(noisy, axis=1, keepdims=True)
        amx = jnp.min(jnp.where(noisy == mx, iota_d, D), axis=1, keepdims=True)
        idxs.append(amx)
        noisy = jnp.where(iota_d == amx, neg, noisy)

    a, b, c, d = idxs
    a, b = jnp.minimum(a, b), jnp.maximum(a, b)
    c, d = jnp.minimum(c, d), jnp.maximum(c, d)
    a, c = jnp.minimum(a, c), jnp.maximum(a, c)
    b, d = jnp.minimum(b, d), jnp.maximum(b, d)
    b, c = jnp.minimum(b, c), jnp.maximum(b, c)
    sorted_idx = (a, b, c, d)

    inv_n = jnp.float32(1.0 / N)
    cols = []
    for k in range(K):
        hist_row = jnp.sum((sorted_idx[k] == iota_d).astype(jnp.float32),
                           axis=0, keepdims=True)
        cols.append(jnp.sum(a2 * hist_row, axis=1, keepdims=True) * inv_n)
    y_ref[0] = jnp.concatenate(cols, axis=1)


def kernel(x, noise):
    return pl.pallas_call(
        _tc_body,
        grid=(C,),
        in_specs=[
            pl.BlockSpec((1, 1, D), lambda c: (c, 0, 0)),
            pl.BlockSpec((1, D, 1), lambda c: (c, 0, 0)),
            pl.BlockSpec((1, N, D), lambda c: (c, 0, 0)),
        ],
        out_specs=pl.BlockSpec((1, D, K), lambda c: (c, 0, 0)),
        out_shape=jax.ShapeDtypeStruct((C, D, K), jnp.float32),
    )(x.reshape(C, 1, D), x.reshape(C, D, 1), noise)
